# baseline (speedup 1.0000x reference)
import jax
import jax.numpy as jnp
from jax.experimental import pallas as pl
from jax.experimental.pallas import tpu as pltpu

_BI = 1024
_BK = 512


def _sage_block(adj_ref, h_ref, wt_ref, out_ref, acc_ref):
    k = pl.program_id(1)
    nk = pl.num_programs(1)
    a = jnp.minimum(adj_ref[...], 1.0)
    p = jnp.dot(a, h_ref[pl.ds(k * _BK, _BK), :],
                preferred_element_type=jnp.float32)

    @pl.when(k == 0)
    def _init():
        acc_ref[...] = p

    @pl.when(k > 0)
    def _accum():
        acc_ref[...] += p

    @pl.when(k == nk - 1)
    def _epilogue():
        out_ref[...] = jnp.dot(acc_ref[...], wt_ref[...],
                               preferred_element_type=jnp.float32)


def kernel(h, adj, W):
    n, d_in = h.shape
    d_out = W.shape[0]
    wt = W.T
    grid = (n // _BI, n // _BK)
    return pl.pallas_call(
        _sage_block,
        grid=grid,
        in_specs=[
            pl.BlockSpec((_BI, _BK), lambda i, k: (i, k)),
            pl.BlockSpec((n, d_in), lambda i, k: (0, 0)),
            pl.BlockSpec((d_in, d_out), lambda i, k: (0, 0)),
        ],
        out_specs=pl.BlockSpec((_BI, d_out), lambda i, k: (i, 0)),
        out_shape=jax.ShapeDtypeStruct((n, d_out), jnp.float32),
        scratch_shapes=[pltpu.VMEM((_BI, d_out), jnp.float32)],
        compiler_params=pltpu.CompilerParams(
            dimension_semantics=("arbitrary", "arbitrary"),
        ),
    )(adj, h, wt)


# speedup vs baseline: 1.5183x; 1.5183x over previous
import jax
import jax.numpy as jnp
from jax.experimental import pallas as pl
from jax.experimental.pallas import tpu as pltpu

_BI = 1024


def _sage_block(adj_ref, h_ref, wt_ref, out_ref, h16_ref, wt16_ref):
    i = pl.program_id(0)

    @pl.when(i == 0)
    def _pack_weights():
        h16_ref[...] = h_ref[...].astype(jnp.bfloat16)
        wt16_ref[...] = wt_ref[...].astype(jnp.bfloat16)

    a16 = jnp.minimum(adj_ref[...], 1.0).astype(jnp.bfloat16)
    x = jnp.dot(a16, h16_ref[...], preferred_element_type=jnp.float32)
    out_ref[...] = jnp.dot(x.astype(jnp.bfloat16), wt16_ref[...],
                           preferred_element_type=jnp.float32)


def kernel(h, adj, W):
    n, d_in = h.shape
    d_out = W.shape[0]
    wt = W.T
    grid = (n // _BI,)
    return pl.pallas_call(
        _sage_block,
        grid=grid,
        in_specs=[
            pl.BlockSpec((_BI, n), lambda i: (i, 0)),
            pl.BlockSpec((n, d_in), lambda i: (0, 0)),
            pl.BlockSpec((d_in, d_out), lambda i: (0, 0)),
        ],
        out_specs=pl.BlockSpec((_BI, d_out), lambda i: (i, 0)),
        out_shape=jax.ShapeDtypeStruct((n, d_out), jnp.float32),
        scratch_shapes=[
            pltpu.VMEM((n, d_in), jnp.bfloat16),
            pltpu.VMEM((d_in, d_out), jnp.bfloat16),
        ],
        compiler_params=pltpu.CompilerParams(
            dimension_semantics=("arbitrary",),
        ),
    )(adj, h, wt)
